# Initial kernel scaffold; baseline (speedup 1.0000x reference)
import jax
import jax.numpy as jnp
from jax.experimental import pallas as pl


def kernel(x, router, router_bias, w_gate_up, w_down):
    raise NotImplementedError("write your pallas kernel here")


# speedup vs baseline: 1.0142x; 1.0142x over previous
import functools

import jax
import jax.numpy as jnp
from jax.experimental import pallas as pl
from jax.experimental.pallas import tpu as pltpu

BT = 128
T = 2048
D = 1024
I = 512
E = 16
K = 2
PAD_T = 4096 + E * BT
G = PAD_T // BT


def _meta_kernel(x_ref, r_ref, b_ref, tok_i_ref, tok_f_ref, small_i_ref,
                 small_f_ref):
    x = x_ref[...]
    logits = jnp.dot(x, r_ref[...], preferred_element_type=jnp.float32)
    biased = logits + b_ref[...]
    lane = jax.lax.broadcasted_iota(jnp.int32, (T, E), 1)
    m0 = jnp.max(biased, axis=1, keepdims=True)
    e0 = jnp.min(jnp.where(biased == m0, lane, jnp.int32(1 << 30)), axis=1,
                 keepdims=True)
    oh0 = (lane == e0).astype(jnp.float32)
    masked = jnp.where(oh0 > 0, jnp.float32(-1e30), biased)
    m1 = jnp.max(masked, axis=1, keepdims=True)
    e1 = jnp.min(jnp.where(masked == m1, lane, jnp.int32(1 << 30)), axis=1,
                 keepdims=True)
    oh1 = (lane == e1).astype(jnp.float32)
    l0 = jnp.sum(logits * oh0, axis=1, keepdims=True)
    l1 = jnp.sum(logits * oh1, axis=1, keepdims=True)
    s0 = jax.nn.sigmoid(l0)
    s1 = jax.nn.sigmoid(l1)
    den = s0 + s1
    tok_f_ref[:, 0:1] = s0 / den
    tok_f_ref[:, 1:2] = s1 / den
    counts0 = jnp.sum(oh0, axis=0, keepdims=True)
    counts1 = jnp.sum(oh1, axis=0, keepdims=True)
    counts = counts0 + counts1
    total = jnp.maximum(jnp.sum(counts), 1.0)
    frac = counts / total
    ent = -jnp.sum(frac * jnp.log(frac + 1e-6))
    small_f_ref[0:1, 0:E] = counts
    small_f_ref[1:2, :] = jnp.full((1, 128), ent, jnp.float32)
    ri = jax.lax.broadcasted_iota(jnp.int32, (T, T), 0)
    ci = jax.lax.broadcasted_iota(jnp.int32, (T, T), 1)
    Lm = (ci < ri).astype(jnp.float32)
    cs0 = jnp.dot(Lm, oh0, preferred_element_type=jnp.float32)
    cs1 = jnp.dot(Lm, oh1, preferred_element_type=jnp.float32)
    rank0 = jnp.sum(cs0 * oh0, axis=1, keepdims=True)
    rank1 = (jnp.sum(cs1 * oh1, axis=1, keepdims=True)
             + jnp.sum(counts0 * oh1, axis=1, keepdims=True))
    pc = jnp.ceil(counts / BT) * BT
    ui = jax.lax.broadcasted_iota(jnp.int32, (E, E), 0)
    uj = jax.lax.broadcasted_iota(jnp.int32, (E, E), 1)
    Um = (ui < uj).astype(jnp.float32)
    po = jnp.dot(pc, Um, preferred_element_type=jnp.float32)
    p0 = jnp.sum(po * oh0, axis=1, keepdims=True) + rank0
    p1 = jnp.sum(po * oh1, axis=1, keepdims=True) + rank1
    tok_i_ref[:, 0:1] = p0.astype(jnp.int32)
    tok_i_ref[:, 1:2] = p1.astype(jnp.int32)
    ends = po + pc
    qcol = (jax.lax.broadcasted_iota(jnp.int32, (64, 1), 0) * BT).astype(
        jnp.float32)
    be = jnp.sum((ends <= qcol).astype(jnp.float32), axis=1, keepdims=True)
    be = jnp.clip(be, 0.0, float(E - 1))
    small_i_ref[:, 0:1] = be.astype(jnp.int32)


def _dispatch_kernel(p0_ref, p1_ref, x_ref, xs_ref):
    g = pl.program_id(0)

    def body(i, carry):
        t = g * BT + i
        row = x_ref[pl.ds(i, 1), :]
        xs_ref[pl.ds(p0_ref[t], 1), :] = row
        xs_ref[pl.ds(p1_ref[t], 1), :] = row
        return carry

    jax.lax.fori_loop(0, BT, body, 0)


def _expert_kernel(be_ref, xs_ref, wgu_ref, wd_ref, y_ref):
    del be_ref
    xg = xs_ref[...]
    gu = jnp.dot(xg, wgu_ref[0], preferred_element_type=jnp.float32)
    gate = gu[:, :I]
    up = gu[:, I:]
    h = gate * jax.nn.sigmoid(gate) * up
    y_ref[...] = jnp.dot(h, wd_ref[0], preferred_element_type=jnp.float32)


def _combine_kernel(p0_ref, p1_ref, y_ref, tf_ref, out_ref):
    g = pl.program_id(0)

    def body(i, carry):
        t = g * BT + i
        r0 = y_ref[pl.ds(p0_ref[t], 1), :]
        r1 = y_ref[pl.ds(p1_ref[t], 1), :]
        w0 = tf_ref[pl.ds(i, 1), 0:1]
        w1 = tf_ref[pl.ds(i, 1), 1:2]
        out_ref[pl.ds(i, 1), :] = w0 * r0 + w1 * r1
        return carry

    jax.lax.fori_loop(0, BT, body, 0)


def kernel(x, router, router_bias, w_gate_up, w_down):
    b, s, d = x.shape
    x_flat = x.reshape(b * s, d)

    tok_i, tok_f, small_i, small_f = pl.pallas_call(
        _meta_kernel,
        out_shape=[
            jax.ShapeDtypeStruct((T, 128), jnp.int32),
            jax.ShapeDtypeStruct((T, 128), jnp.float32),
            jax.ShapeDtypeStruct((64, 128), jnp.int32),
            jax.ShapeDtypeStruct((8, 128), jnp.float32),
        ],
    )(x_flat, router, router_bias.reshape(1, E))

    p0 = tok_i[:, 0]
    p1 = tok_i[:, 1]
    be = small_i[:G, 0]
    expert_counts = small_f[0, :E]
    entropy = small_f[1, 0]

    xs = pl.pallas_call(
        _dispatch_kernel,
        grid_spec=pltpu.PrefetchScalarGridSpec(
            num_scalar_prefetch=2,
            grid=(T // BT,),
            in_specs=[pl.BlockSpec((BT, D), lambda g, p0, p1: (g, 0))],
            out_specs=pl.BlockSpec((PAD_T, D), lambda g, p0, p1: (0, 0)),
        ),
        out_shape=jax.ShapeDtypeStruct((PAD_T, D), jnp.float32),
    )(p0, p1, x_flat)

    y = pl.pallas_call(
        _expert_kernel,
        grid_spec=pltpu.PrefetchScalarGridSpec(
            num_scalar_prefetch=1,
            grid=(G,),
            in_specs=[
                pl.BlockSpec((BT, D), lambda g, be: (g, 0)),
                pl.BlockSpec((1, D, 2 * I), lambda g, be: (be[g], 0, 0)),
                pl.BlockSpec((1, I, D), lambda g, be: (be[g], 0, 0)),
            ],
            out_specs=pl.BlockSpec((BT, D), lambda g, be: (g, 0)),
        ),
        out_shape=jax.ShapeDtypeStruct((PAD_T, D), jnp.float32),
    )(be, xs, w_gate_up, w_down)

    routed = pl.pallas_call(
        _combine_kernel,
        grid_spec=pltpu.PrefetchScalarGridSpec(
            num_scalar_prefetch=2,
            grid=(T // BT,),
            in_specs=[
                pl.BlockSpec((PAD_T, D), lambda g, p0, p1: (0, 0)),
                pl.BlockSpec((BT, 128), lambda g, p0, p1: (g, 0)),
            ],
            out_specs=pl.BlockSpec((BT, D), lambda g, p0, p1: (g, 0)),
        ),
        out_shape=jax.ShapeDtypeStruct((T, D), jnp.float32),
    )(p0, p1, y, tok_f)

    return routed.reshape(b, s, d), expert_counts, entropy


# speedup vs baseline: 1.9604x; 1.9329x over previous
import functools

import jax
import jax.numpy as jnp
from jax import lax
from jax.experimental import pallas as pl
from jax.experimental.pallas import tpu as pltpu
from jax.experimental.pallas import tpu_sc as plsc

BT = 128
T = 2048
D = 1024
I = 512
E = 16
K = 2
PAD_T = 4096 + E * BT
G = PAD_T // BT

NC = 2
NS = 16
NW = NC * NS
L = 16


def _meta_kernel(x_ref, r_ref, b_ref, tok_i_ref, tok_f_ref, small_i_ref,
                 small_f_ref):
    x = x_ref[...]
    logits = jnp.dot(x, r_ref[...], preferred_element_type=jnp.float32)
    biased = logits + b_ref[...]
    lane = jax.lax.broadcasted_iota(jnp.int32, (T, E), 1)
    m0 = jnp.max(biased, axis=1, keepdims=True)
    e0 = jnp.min(jnp.where(biased == m0, lane, jnp.int32(1 << 30)), axis=1,
                 keepdims=True)
    oh0 = (lane == e0).astype(jnp.float32)
    masked = jnp.where(oh0 > 0, jnp.float32(-1e30), biased)
    m1 = jnp.max(masked, axis=1, keepdims=True)
    e1 = jnp.min(jnp.where(masked == m1, lane, jnp.int32(1 << 30)), axis=1,
                 keepdims=True)
    oh1 = (lane == e1).astype(jnp.float32)
    l0 = jnp.sum(logits * oh0, axis=1, keepdims=True)
    l1 = jnp.sum(logits * oh1, axis=1, keepdims=True)
    s0 = jax.nn.sigmoid(l0)
    s1 = jax.nn.sigmoid(l1)
    den = s0 + s1
    tok_f_ref[:, 0:1] = s0 / den
    tok_f_ref[:, 1:2] = s1 / den
    counts0 = jnp.sum(oh0, axis=0, keepdims=True)
    counts1 = jnp.sum(oh1, axis=0, keepdims=True)
    counts = counts0 + counts1
    total = jnp.maximum(jnp.sum(counts), 1.0)
    frac = counts / total
    ent = -jnp.sum(frac * jnp.log(frac + 1e-6))
    small_f_ref[0:1, 0:E] = counts
    small_f_ref[1:2, :] = jnp.full((1, 128), ent, jnp.float32)
    ri = jax.lax.broadcasted_iota(jnp.int32, (T, T), 0)
    ci = jax.lax.broadcasted_iota(jnp.int32, (T, T), 1)
    Lm = (ci < ri).astype(jnp.float32)
    cs0 = jnp.dot(Lm, oh0, preferred_element_type=jnp.float32)
    cs1 = jnp.dot(Lm, oh1, preferred_element_type=jnp.float32)
    rank0 = jnp.sum(cs0 * oh0, axis=1, keepdims=True)
    rank1 = (jnp.sum(cs1 * oh1, axis=1, keepdims=True)
             + jnp.sum(counts0 * oh1, axis=1, keepdims=True))
    pc = jnp.ceil(counts / BT) * BT
    ui = jax.lax.broadcasted_iota(jnp.int32, (E, E), 0)
    uj = jax.lax.broadcasted_iota(jnp.int32, (E, E), 1)
    Um = (ui < uj).astype(jnp.float32)
    po = jnp.dot(pc, Um, preferred_element_type=jnp.float32)
    p0 = jnp.sum(po * oh0, axis=1, keepdims=True) + rank0
    p1 = jnp.sum(po * oh1, axis=1, keepdims=True) + rank1
    tok_i_ref[:, 0:1] = p0.astype(jnp.int32)
    tok_i_ref[:, 1:2] = p1.astype(jnp.int32)
    ends = po + pc
    qcol = (jax.lax.broadcasted_iota(jnp.int32, (64, 1), 0) * BT).astype(
        jnp.float32)
    be = jnp.sum((ends <= qcol).astype(jnp.float32), axis=1, keepdims=True)
    be = jnp.clip(be, 0.0, float(E - 1))
    small_i_ref[:, 0:1] = be.astype(jnp.int32)


def _sc_wid():
    return lax.axis_index("s") * NC + lax.axis_index("c")


_W_TOK = T // NW


@functools.partial(
    pl.kernel,
    mesh=plsc.VectorSubcoreMesh(core_axis_name="c", subcore_axis_name="s"),
    out_type=jax.ShapeDtypeStruct((PAD_T, D), jnp.float32),
    scratch_types=[
        pltpu.VMEM((K, _W_TOK), jnp.int32),
        pltpu.VMEM((_W_TOK, D), jnp.float32),
        pltpu.SemaphoreType.DMA,
        pltpu.SemaphoreType.DMA,
    ],
)
def _dispatch_sc(pidx_hbm, x_hbm, xs_hbm, idx_v, rows_v, sem0, sem1):
    wid = _sc_wid()
    base = wid * _W_TOK
    pltpu.sync_copy(pidx_hbm.at[wid], idx_v)
    pltpu.sync_copy(x_hbm.at[pl.ds(base, _W_TOK)], rows_v)
    c0 = pltpu.async_copy(rows_v, xs_hbm.at[idx_v.at[0]], sem0)
    c1 = pltpu.async_copy(rows_v, xs_hbm.at[idx_v.at[1]], sem1)
    c0.wait()
    c1.wait()


def _expert_kernel(be_ref, xs_ref, wgu_ref, wd_ref, y_ref):
    del be_ref
    xg = xs_ref[...]
    gu = jnp.dot(xg, wgu_ref[0], preferred_element_type=jnp.float32)
    gate = gu[:, :I]
    up = gu[:, I:]
    h = gate * jax.nn.sigmoid(gate) * up
    y_ref[...] = jnp.dot(h, wd_ref[0], preferred_element_type=jnp.float32)


_C_TOK = T // NW
_C_CH = 32
_C_NCH = _C_TOK // _C_CH


@functools.partial(
    pl.kernel,
    mesh=plsc.VectorSubcoreMesh(core_axis_name="c", subcore_axis_name="s"),
    out_type=[
        jax.ShapeDtypeStruct((T, D), jnp.float32),
        jax.ShapeDtypeStruct((T, D), jnp.float32),
    ],
    scratch_types=[
        pltpu.VMEM((_C_TOK,), jnp.int32),
        pltpu.VMEM((_C_TOK,), jnp.int32),
        pltpu.VMEM((_C_CH, D), jnp.float32),
        pltpu.VMEM((_C_CH, D), jnp.float32),
        pltpu.SemaphoreType.DMA,
        pltpu.SemaphoreType.DMA,
    ],
)
def _combine_g_sc(p0_hbm, p1_hbm, y_hbm, y0_hbm, y1_hbm, i0_v, i1_v, buf0,
                  buf1, sem0, sem1):
    wid = _sc_wid()
    base = wid * _C_TOK
    pltpu.sync_copy(p0_hbm.at[pl.ds(base, _C_TOK)], i0_v)
    pltpu.sync_copy(p1_hbm.at[pl.ds(base, _C_TOK)], i1_v)
    bufs = (buf0, buf1)
    sems = (sem0, sem1)
    plan = ([(i0_v, y0_hbm, c) for c in range(_C_NCH)]
            + [(i1_v, y1_hbm, c) for c in range(_C_NCH)])
    n = len(plan)
    iv, _, c = plan[0]
    prev = pltpu.async_copy(y_hbm.at[iv.at[pl.ds(c * _C_CH, _C_CH)]], bufs[0],
                            sems[0])
    for k in range(1, n):
        iv, _, c = plan[k]
        cur = pltpu.async_copy(y_hbm.at[iv.at[pl.ds(c * _C_CH, _C_CH)]],
                               bufs[k % 2], sems[k % 2])
        prev.wait()
        _, dst, cc = plan[k - 1]
        pltpu.sync_copy(bufs[(k - 1) % 2],
                        dst.at[pl.ds(base + cc * _C_CH, _C_CH)])
        prev = cur
    prev.wait()
    _, dst, cc = plan[n - 1]
    pltpu.sync_copy(bufs[(n - 1) % 2], dst.at[pl.ds(base + cc * _C_CH, _C_CH)])


def _combine_w_kernel(y0_ref, y1_ref, tf_ref, out_ref):
    w0 = tf_ref[:, 0:1]
    w1 = tf_ref[:, 1:2]
    out_ref[...] = w0 * y0_ref[...] + w1 * y1_ref[...]


def kernel(x, router, router_bias, w_gate_up, w_down):
    b, s, d = x.shape
    x_flat = x.reshape(b * s, d)

    tok_i, tok_f, small_i, small_f = pl.pallas_call(
        _meta_kernel,
        out_shape=[
            jax.ShapeDtypeStruct((T, 128), jnp.int32),
            jax.ShapeDtypeStruct((T, 128), jnp.float32),
            jax.ShapeDtypeStruct((64, 128), jnp.int32),
            jax.ShapeDtypeStruct((8, 128), jnp.float32),
        ],
    )(x_flat, router, router_bias.reshape(1, E))

    p0 = tok_i[:, 0]
    p1 = tok_i[:, 1]
    be = small_i[:G, 0]
    expert_counts = small_f[0, :E]
    entropy = small_f[1, 0]

    pidx = jnp.stack([p0.reshape(NW, _W_TOK), p1.reshape(NW, _W_TOK)], axis=1)
    xs = _dispatch_sc(pidx, x_flat)

    y = pl.pallas_call(
        _expert_kernel,
        grid_spec=pltpu.PrefetchScalarGridSpec(
            num_scalar_prefetch=1,
            grid=(G,),
            in_specs=[
                pl.BlockSpec((BT, D), lambda g, be: (g, 0)),
                pl.BlockSpec((1, D, 2 * I), lambda g, be: (be[g], 0, 0)),
                pl.BlockSpec((1, I, D), lambda g, be: (be[g], 0, 0)),
            ],
            out_specs=pl.BlockSpec((BT, D), lambda g, be: (g, 0)),
        ),
        out_shape=jax.ShapeDtypeStruct((PAD_T, D), jnp.float32),
    )(be, xs, w_gate_up, w_down)

    y0, y1 = _combine_g_sc(p0, p1, y)

    routed = pl.pallas_call(
        _combine_w_kernel,
        grid=(T // BT,),
        in_specs=[
            pl.BlockSpec((BT, D), lambda g: (g, 0)),
            pl.BlockSpec((BT, D), lambda g: (g, 0)),
            pl.BlockSpec((BT, 128), lambda g: (g, 0)),
        ],
        out_specs=pl.BlockSpec((BT, D), lambda g: (g, 0)),
        out_shape=jax.ShapeDtypeStruct((T, D), jnp.float32),
    )(y0, y1, tok_f)

    return routed.reshape(b, s, d), expert_counts, entropy


# speedup vs baseline: 1.9945x; 1.0174x over previous
import functools

import jax
import jax.numpy as jnp
from jax import lax
from jax.experimental import pallas as pl
from jax.experimental.pallas import tpu as pltpu
from jax.experimental.pallas import tpu_sc as plsc

BT = 128
T = 2048
D = 1024
I = 512
E = 16
K = 2
PAD_T = 4096 + E * BT
G = PAD_T // BT

NC = 2
NS = 16
NW = NC * NS
L = 16


def _meta_kernel(x_ref, r_ref, b_ref, tok_i_ref, tok_f_ref, small_i_ref,
                 small_f_ref):
    x = x_ref[...]
    logits = jnp.dot(x, r_ref[...], preferred_element_type=jnp.float32)
    biased = logits + b_ref[...]
    lane = jax.lax.broadcasted_iota(jnp.int32, (T, E), 1)
    m0 = jnp.max(biased, axis=1, keepdims=True)
    e0 = jnp.min(jnp.where(biased == m0, lane, jnp.int32(1 << 30)), axis=1,
                 keepdims=True)
    oh0 = (lane == e0).astype(jnp.float32)
    masked = jnp.where(oh0 > 0, jnp.float32(-1e30), biased)
    m1 = jnp.max(masked, axis=1, keepdims=True)
    e1 = jnp.min(jnp.where(masked == m1, lane, jnp.int32(1 << 30)), axis=1,
                 keepdims=True)
    oh1 = (lane == e1).astype(jnp.float32)
    l0 = jnp.sum(logits * oh0, axis=1, keepdims=True)
    l1 = jnp.sum(logits * oh1, axis=1, keepdims=True)
    s0 = jax.nn.sigmoid(l0)
    s1 = jax.nn.sigmoid(l1)
    den = s0 + s1
    tok_f_ref[:, 0:1] = s0 / den
    tok_f_ref[:, 1:2] = s1 / den
    counts0 = jnp.sum(oh0, axis=0, keepdims=True)
    counts1 = jnp.sum(oh1, axis=0, keepdims=True)
    counts = counts0 + counts1
    total = jnp.maximum(jnp.sum(counts), 1.0)
    frac = counts / total
    ent = -jnp.sum(frac * jnp.log(frac + 1e-6))
    small_f_ref[0:1, 0:E] = counts
    small_f_ref[1:2, :] = jnp.full((1, 128), ent, jnp.float32)
    ri = jax.lax.broadcasted_iota(jnp.int32, (T, T), 0)
    ci = jax.lax.broadcasted_iota(jnp.int32, (T, T), 1)
    Lm = (ci < ri).astype(jnp.float32)
    cs0 = jnp.dot(Lm, oh0, preferred_element_type=jnp.float32)
    cs1 = jnp.dot(Lm, oh1, preferred_element_type=jnp.float32)
    rank0 = jnp.sum(cs0 * oh0, axis=1, keepdims=True)
    rank1 = (jnp.sum(cs1 * oh1, axis=1, keepdims=True)
             + jnp.sum(counts0 * oh1, axis=1, keepdims=True))
    pc = jnp.ceil(counts / BT) * BT
    ui = jax.lax.broadcasted_iota(jnp.int32, (E, E), 0)
    uj = jax.lax.broadcasted_iota(jnp.int32, (E, E), 1)
    Um = (ui < uj).astype(jnp.float32)
    po = jnp.dot(pc, Um, preferred_element_type=jnp.float32)
    p0 = jnp.sum(po * oh0, axis=1, keepdims=True) + rank0
    p1 = jnp.sum(po * oh1, axis=1, keepdims=True) + rank1
    tok_i_ref[0:1, :] = jnp.transpose(p0.astype(jnp.int32))
    tok_i_ref[1:2, :] = jnp.transpose(p1.astype(jnp.int32))
    ends = po + pc
    qcol = (jax.lax.broadcasted_iota(jnp.int32, (64, 1), 0) * BT).astype(
        jnp.float32)
    be = jnp.sum((ends <= qcol).astype(jnp.float32), axis=1, keepdims=True)
    be = jnp.clip(be, 0.0, float(E - 1))
    small_i_ref[:, 0:1] = be.astype(jnp.int32)


def _sc_wid():
    return lax.axis_index("s") * NC + lax.axis_index("c")


_W_TOK = T // NW


@functools.partial(
    pl.kernel,
    mesh=plsc.VectorSubcoreMesh(core_axis_name="c", subcore_axis_name="s"),
    out_type=jax.ShapeDtypeStruct((PAD_T, D), jnp.float32),
    scratch_types=[
        pltpu.VMEM((K, _W_TOK), jnp.int32),
        pltpu.VMEM((_W_TOK, D), jnp.float32),
        pltpu.SemaphoreType.DMA,
        pltpu.SemaphoreType.DMA,
    ],
)
def _dispatch_sc(pos_hbm, x_hbm, xs_hbm, idx_v, rows_v, sem0, sem1):
    wid = _sc_wid()
    base = wid * _W_TOK
    pltpu.sync_copy(pos_hbm.at[0, pl.ds(base, _W_TOK)], idx_v.at[0])
    pltpu.sync_copy(pos_hbm.at[1, pl.ds(base, _W_TOK)], idx_v.at[1])
    pltpu.sync_copy(x_hbm.at[pl.ds(base, _W_TOK)], rows_v)
    c0 = pltpu.async_copy(rows_v, xs_hbm.at[idx_v.at[0]], sem0)
    c1 = pltpu.async_copy(rows_v, xs_hbm.at[idx_v.at[1]], sem1)
    c0.wait()
    c1.wait()


def _expert_kernel(be_ref, xs_ref, wgu_ref, wd_ref, y_ref):
    del be_ref
    xg = xs_ref[...]
    gu = jnp.dot(xg, wgu_ref[0], preferred_element_type=jnp.float32)
    gate = gu[:, :I]
    up = gu[:, I:]
    h = gate * jax.nn.sigmoid(gate) * up
    y_ref[...] = jnp.dot(h, wd_ref[0], preferred_element_type=jnp.float32)


_C_TOK = T // NW
_C_CH = 32
_C_NCH = _C_TOK // _C_CH


@functools.partial(
    pl.kernel,
    mesh=plsc.VectorSubcoreMesh(core_axis_name="c", subcore_axis_name="s"),
    out_type=[
        jax.ShapeDtypeStruct((T, D), jnp.float32),
        jax.ShapeDtypeStruct((T, D), jnp.float32),
    ],
    scratch_types=[
        pltpu.VMEM((_C_TOK,), jnp.int32),
        pltpu.VMEM((_C_TOK,), jnp.int32),
        pltpu.VMEM((_C_CH, D), jnp.float32),
        pltpu.VMEM((_C_CH, D), jnp.float32),
        pltpu.SemaphoreType.DMA,
        pltpu.SemaphoreType.DMA,
    ],
)
def _combine_g_sc(pos_hbm, y_hbm, y0_hbm, y1_hbm, i0_v, i1_v, buf0,
                  buf1, sem0, sem1):
    wid = _sc_wid()
    base = wid * _C_TOK
    pltpu.sync_copy(pos_hbm.at[0, pl.ds(base, _C_TOK)], i0_v)
    pltpu.sync_copy(pos_hbm.at[1, pl.ds(base, _C_TOK)], i1_v)
    bufs = (buf0, buf1)
    sems = (sem0, sem1)
    plan = ([(i0_v, y0_hbm, c) for c in range(_C_NCH)]
            + [(i1_v, y1_hbm, c) for c in range(_C_NCH)])
    n = len(plan)
    iv, _, c = plan[0]
    prev = pltpu.async_copy(y_hbm.at[iv.at[pl.ds(c * _C_CH, _C_CH)]], bufs[0],
                            sems[0])
    for k in range(1, n):
        iv, _, c = plan[k]
        cur = pltpu.async_copy(y_hbm.at[iv.at[pl.ds(c * _C_CH, _C_CH)]],
                               bufs[k % 2], sems[k % 2])
        prev.wait()
        _, dst, cc = plan[k - 1]
        pltpu.sync_copy(bufs[(k - 1) % 2],
                        dst.at[pl.ds(base + cc * _C_CH, _C_CH)])
        prev = cur
    prev.wait()
    _, dst, cc = plan[n - 1]
    pltpu.sync_copy(bufs[(n - 1) % 2], dst.at[pl.ds(base + cc * _C_CH, _C_CH)])


def _combine_w_kernel(y0_ref, y1_ref, tf_ref, out_ref):
    w0 = tf_ref[:, 0:1]
    w1 = tf_ref[:, 1:2]
    out_ref[...] = w0 * y0_ref[...] + w1 * y1_ref[...]


def kernel(x, router, router_bias, w_gate_up, w_down):
    b, s, d = x.shape
    x_flat = x.reshape(b * s, d)

    pos, tok_f, small_i, small_f = pl.pallas_call(
        _meta_kernel,
        out_shape=[
            jax.ShapeDtypeStruct((8, T), jnp.int32),
            jax.ShapeDtypeStruct((T, 128), jnp.float32),
            jax.ShapeDtypeStruct((64, 8), jnp.int32),
            jax.ShapeDtypeStruct((8, 128), jnp.float32),
        ],
    )(x_flat, router, router_bias.reshape(1, E))

    expert_counts = small_f[0, :E]
    entropy = small_f[1, 0]

    xs = _dispatch_sc(pos, x_flat)

    y = pl.pallas_call(
        _expert_kernel,
        grid_spec=pltpu.PrefetchScalarGridSpec(
            num_scalar_prefetch=1,
            grid=(G,),
            in_specs=[
                pl.BlockSpec((BT, D), lambda g, be: (g, 0)),
                pl.BlockSpec((1, D, 2 * I), lambda g, be: (be[g, 0], 0, 0)),
                pl.BlockSpec((1, I, D), lambda g, be: (be[g, 0], 0, 0)),
            ],
            out_specs=pl.BlockSpec((BT, D), lambda g, be: (g, 0)),
        ),
        out_shape=jax.ShapeDtypeStruct((PAD_T, D), jnp.float32),
    )(small_i, xs, w_gate_up, w_down)

    y0, y1 = _combine_g_sc(pos, y)

    routed = pl.pallas_call(
        _combine_w_kernel,
        grid=(T // BT,),
        in_specs=[
            pl.BlockSpec((BT, D), lambda g: (g, 0)),
            pl.BlockSpec((BT, D), lambda g: (g, 0)),
            pl.BlockSpec((BT, 128), lambda g: (g, 0)),
        ],
        out_specs=pl.BlockSpec((BT, D), lambda g: (g, 0)),
        out_shape=jax.ShapeDtypeStruct((T, D), jnp.float32),
    )(y0, y1, tok_f)

    return routed.reshape(b, s, d), expert_counts, entropy


# speedup vs baseline: 2.1116x; 1.0587x over previous
import functools

import jax
import jax.numpy as jnp
from jax import lax
from jax.experimental import pallas as pl
from jax.experimental.pallas import tpu as pltpu
from jax.experimental.pallas import tpu_sc as plsc

BT = 128
T = 2048
D = 1024
I = 512
E = 16
K = 2
PAD_T = 4096 + E * BT
G = PAD_T // BT

NC = 2
NS = 16
NW = NC * NS
L = 16


def _meta_kernel(x_ref, r_ref, b_ref, tok_i_ref, w0e_ref, w1e_ref,
                 small_i_ref, small_f_ref):
    x = x_ref[...]
    logits = jnp.dot(x, r_ref[...], preferred_element_type=jnp.float32)
    biased = logits + b_ref[...]
    lane = jax.lax.broadcasted_iota(jnp.int32, (T, E), 1)
    m0 = jnp.max(biased, axis=1, keepdims=True)
    e0 = jnp.min(jnp.where(biased == m0, lane, jnp.int32(1 << 30)), axis=1,
                 keepdims=True)
    oh0 = (lane == e0).astype(jnp.float32)
    masked = jnp.where(oh0 > 0, jnp.float32(-1e30), biased)
    m1 = jnp.max(masked, axis=1, keepdims=True)
    e1 = jnp.min(jnp.where(masked == m1, lane, jnp.int32(1 << 30)), axis=1,
                 keepdims=True)
    oh1 = (lane == e1).astype(jnp.float32)
    l0 = jnp.sum(logits * oh0, axis=1, keepdims=True)
    l1 = jnp.sum(logits * oh1, axis=1, keepdims=True)
    s0 = jax.nn.sigmoid(l0)
    s1 = jax.nn.sigmoid(l1)
    den = s0 + s1
    w0e_ref[...] = jnp.broadcast_to(s0 / den, (T, L))
    w1e_ref[...] = jnp.broadcast_to(s1 / den, (T, L))
    counts0 = jnp.sum(oh0, axis=0, keepdims=True)
    counts1 = jnp.sum(oh1, axis=0, keepdims=True)
    counts = counts0 + counts1
    total = jnp.maximum(jnp.sum(counts), 1.0)
    frac = counts / total
    ent = -jnp.sum(frac * jnp.log(frac + 1e-6))
    small_f_ref[0:1, 0:E] = counts
    small_f_ref[1:2, :] = jnp.full((1, 128), ent, jnp.float32)
    ri = jax.lax.broadcasted_iota(jnp.int32, (T, T), 0)
    ci = jax.lax.broadcasted_iota(jnp.int32, (T, T), 1)
    Lm = (ci < ri).astype(jnp.float32)
    cs0 = jnp.dot(Lm, oh0, preferred_element_type=jnp.float32)
    cs1 = jnp.dot(Lm, oh1, preferred_element_type=jnp.float32)
    rank0 = jnp.sum(cs0 * oh0, axis=1, keepdims=True)
    rank1 = (jnp.sum(cs1 * oh1, axis=1, keepdims=True)
             + jnp.sum(counts0 * oh1, axis=1, keepdims=True))
    pc = jnp.ceil(counts / BT) * BT
    ui = jax.lax.broadcasted_iota(jnp.int32, (E, E), 0)
    uj = jax.lax.broadcasted_iota(jnp.int32, (E, E), 1)
    Um = (ui < uj).astype(jnp.float32)
    po = jnp.dot(pc, Um, preferred_element_type=jnp.float32)
    p0 = jnp.sum(po * oh0, axis=1, keepdims=True) + rank0
    p1 = jnp.sum(po * oh1, axis=1, keepdims=True) + rank1
    tok_i_ref[0:1, :] = jnp.transpose(p0.astype(jnp.int32))
    tok_i_ref[1:2, :] = jnp.transpose(p1.astype(jnp.int32))
    ends = po + pc
    qcol = (jax.lax.broadcasted_iota(jnp.int32, (64, 1), 0) * BT).astype(
        jnp.float32)
    be = jnp.sum((ends <= qcol).astype(jnp.float32), axis=1, keepdims=True)
    be = jnp.clip(be, 0.0, float(E - 1))
    small_i_ref[:, 0:1] = be.astype(jnp.int32)


def _sc_wid():
    return lax.axis_index("s") * NC + lax.axis_index("c")


_W_TOK = T // NW


@functools.partial(
    pl.kernel,
    mesh=plsc.VectorSubcoreMesh(core_axis_name="c", subcore_axis_name="s"),
    out_type=jax.ShapeDtypeStruct((PAD_T, D), jnp.float32),
    scratch_types=[
        pltpu.VMEM((K, _W_TOK), jnp.int32),
        pltpu.VMEM((_W_TOK, D), jnp.float32),
        pltpu.SemaphoreType.DMA,
        pltpu.SemaphoreType.DMA,
    ],
)
def _dispatch_sc(pos_hbm, x_hbm, xs_hbm, idx_v, rows_v, sem0, sem1):
    wid = _sc_wid()
    base = wid * _W_TOK
    pltpu.sync_copy(pos_hbm.at[0, pl.ds(base, _W_TOK)], idx_v.at[0])
    pltpu.sync_copy(pos_hbm.at[1, pl.ds(base, _W_TOK)], idx_v.at[1])
    pltpu.sync_copy(x_hbm.at[pl.ds(base, _W_TOK)], rows_v)
    c0 = pltpu.async_copy(rows_v, xs_hbm.at[idx_v.at[0]], sem0)
    c1 = pltpu.async_copy(rows_v, xs_hbm.at[idx_v.at[1]], sem1)
    c0.wait()
    c1.wait()


def _expert_kernel(be_ref, xs_ref, wgu_ref, wd_ref, y_ref):
    del be_ref
    xg = xs_ref[...]
    gu = jnp.dot(xg, wgu_ref[0], preferred_element_type=jnp.float32)
    gate = gu[:, :I]
    up = gu[:, I:]
    h = gate * jax.nn.sigmoid(gate) * up
    y_ref[...] = jnp.dot(h, wd_ref[0], preferred_element_type=jnp.float32)


_C_TOK = T // NW
_C_CH = 32
_C_NCH = _C_TOK // _C_CH


@functools.partial(
    pl.kernel,
    mesh=plsc.VectorSubcoreMesh(core_axis_name="c", subcore_axis_name="s"),
    out_type=jax.ShapeDtypeStruct((T, D), jnp.float32),
    scratch_types=[
        pltpu.VMEM((K, _C_TOK), jnp.int32),
        pltpu.VMEM((_C_TOK, L), jnp.float32),
        pltpu.VMEM((_C_TOK, L), jnp.float32),
        pltpu.VMEM((_C_CH, D), jnp.float32),
        pltpu.VMEM((_C_CH, D), jnp.float32),
        pltpu.VMEM((_C_CH, D), jnp.float32),
        pltpu.SemaphoreType.DMA,
        pltpu.SemaphoreType.DMA,
    ],
)
def _combine_sc(pos_hbm, w0_hbm, w1_hbm, y_hbm, out_hbm, idx_v, w0_v, w1_v,
                buf0, buf1, obuf, sem0, sem1):
    wid = _sc_wid()
    base = wid * _C_TOK
    pltpu.sync_copy(pos_hbm.at[0, pl.ds(base, _C_TOK)], idx_v.at[0])
    pltpu.sync_copy(pos_hbm.at[1, pl.ds(base, _C_TOK)], idx_v.at[1])
    pltpu.sync_copy(w0_hbm.at[pl.ds(base, _C_TOK)], w0_v)
    pltpu.sync_copy(w1_hbm.at[pl.ds(base, _C_TOK)], w1_v)
    for c in range(_C_NCH):
        c0 = pltpu.async_copy(y_hbm.at[idx_v.at[0, pl.ds(c * _C_CH, _C_CH)]],
                              buf0, sem0)
        c1 = pltpu.async_copy(y_hbm.at[idx_v.at[1, pl.ds(c * _C_CH, _C_CH)]],
                              buf1, sem1)
        c0.wait()
        c1.wait()

        def tok(t, carry):
            w0b = w0_v[c * _C_CH + t, :]
            w1b = w1_v[c * _C_CH + t, :]
            for u in range(D // L):
                obuf[t, pl.ds(u * L, L)] = (
                    buf0[t, pl.ds(u * L, L)] * w0b
                    + buf1[t, pl.ds(u * L, L)] * w1b)
            return carry

        lax.fori_loop(0, _C_CH, tok, 0)
        pltpu.sync_copy(obuf, out_hbm.at[pl.ds(base + c * _C_CH, _C_CH)])


def kernel(x, router, router_bias, w_gate_up, w_down):
    b, s, d = x.shape
    x_flat = x.reshape(b * s, d)

    pos, w0e, w1e, small_i, small_f = pl.pallas_call(
        _meta_kernel,
        out_shape=[
            jax.ShapeDtypeStruct((8, T), jnp.int32),
            jax.ShapeDtypeStruct((T, L), jnp.float32),
            jax.ShapeDtypeStruct((T, L), jnp.float32),
            jax.ShapeDtypeStruct((64, 8), jnp.int32),
            jax.ShapeDtypeStruct((8, 128), jnp.float32),
        ],
    )(x_flat, router, router_bias.reshape(1, E))

    expert_counts = small_f[0, :E]
    entropy = small_f[1, 0]

    xs = _dispatch_sc(pos, x_flat)

    y = pl.pallas_call(
        _expert_kernel,
        grid_spec=pltpu.PrefetchScalarGridSpec(
            num_scalar_prefetch=1,
            grid=(G,),
            in_specs=[
                pl.BlockSpec((BT, D), lambda g, be: (g, 0)),
                pl.BlockSpec((1, D, 2 * I), lambda g, be: (be[g, 0], 0, 0)),
                pl.BlockSpec((1, I, D), lambda g, be: (be[g, 0], 0, 0)),
            ],
            out_specs=pl.BlockSpec((BT, D), lambda g, be: (g, 0)),
        ),
        out_shape=jax.ShapeDtypeStruct((PAD_T, D), jnp.float32),
    )(small_i, xs, w_gate_up, w_down)

    routed = _combine_sc(pos, w0e, w1e, y)

    return routed.reshape(b, s, d), expert_counts, entropy
